# baseline (speedup 1.0000x reference)
import functools

import jax
import jax.numpy as jnp
from jax import lax
from jax.experimental import pallas as pl
from jax.experimental.pallas import tpu as pltpu
from jax.experimental.pallas import tpu_sc as plsc

N_POINTS = 1_000_000
EMB = 16
OUT_D = EMB + 3
CH = 800
NCHUNK = N_POINTS // CH
NW = 32
MAX_CH_W = -(-NCHUNK // NW)
NBLK = CH // 16

_mesh = plsc.VectorSubcoreMesh(core_axis_name="c", subcore_axis_name="s")

_BCAST_DNUMS = lax.GatherDimensionNumbers(
    offset_dims=(), collapsed_slice_dims=(0,), start_index_map=(0,))


def _lane_bcast(vec, idx_splat):
    return lax.gather(vec, idx_splat[:, None], _BCAST_DNUMS, (1,),
                      mode=lax.GatherScatterMode.PROMISE_IN_BOUNDS)


@functools.partial(
    pl.kernel,
    mesh=_mesh,
    compiler_params=pltpu.CompilerParams(
        needs_layout_passes=False, use_tc_tiling_on_sc=False),
    out_type=jax.ShapeDtypeStruct((N_POINTS * OUT_D,), jnp.float32),
    scratch_types=[
        pltpu.VMEM((CH,), jnp.int32),
        pltpu.VMEM((CH,), jnp.int32),
        pltpu.VMEM((CH,), jnp.int32),
        pltpu.VMEM((CH,), jnp.int32),
        pltpu.VMEM((CH,), jnp.int32),
        pltpu.VMEM((CH,), jnp.int32),
        pltpu.VMEM((CH,), jnp.int32),
        pltpu.VMEM((CH, EMB), jnp.float32),
        pltpu.VMEM((CH, EMB), jnp.float32),
        pltpu.VMEM((CH, EMB), jnp.float32),
        pltpu.VMEM((CH * 3,), jnp.float32),
        pltpu.VMEM((CH * 3,), jnp.float32),
        pltpu.VMEM((CH * OUT_D,), jnp.float32),
        pltpu.SemaphoreType.DMA,
    ],
)
def _encode(x_hbm, fidx_hbm, bary_hbm, emb_hbm, faces_hbm, out_hbm,
            fidx_v, i0_v, i1_v, i2_v, v0_v, v1_v, v2_v,
            e0_v, e1_v, e2_v, bary_v, x_v, out_v, sem):
    wid = lax.axis_index("s") * 2 + lax.axis_index("c")

    def chunk_body(j, carry):
        c = j * NW + wid

        @pl.when(c < NCHUNK)
        def _():
            o = c * CH
            pltpu.sync_copy(fidx_hbm.at[pl.ds(o, CH)], fidx_v)
            pltpu.sync_copy(bary_hbm.at[pl.ds(o * 3, CH * 3)], bary_v)
            pltpu.sync_copy(x_hbm.at[pl.ds(o * 3, CH * 3)], x_v)

            def mkidx(k, carry2):
                sl = pl.ds(k * 16, 16)
                f3 = fidx_v[sl] * 3
                i0_v[sl] = f3
                i1_v[sl] = f3 + 1
                i2_v[sl] = f3 + 2
                return carry2

            lax.fori_loop(0, NBLK, mkidx, 0)

            cpa = pltpu.async_copy(faces_hbm.at[i0_v], v0_v, sem)
            cpb = pltpu.async_copy(faces_hbm.at[i1_v], v1_v, sem)
            cpc = pltpu.async_copy(faces_hbm.at[i2_v], v2_v, sem)
            cpa.wait()
            cpb.wait()
            cpc.wait()

            cp0 = pltpu.async_copy(emb_hbm.at[v0_v], e0_v, sem)
            cp1 = pltpu.async_copy(emb_hbm.at[v1_v], e1_v, sem)
            cp2 = pltpu.async_copy(emb_hbm.at[v2_v], e2_v, sem)
            cp0.wait()
            cp1.wait()
            cp2.wait()

            def blk(k, carry2):
                rows = lax.iota(jnp.int32, 16) + k * 16
                base3 = rows * 3
                w0 = plsc.load_gather(bary_v, [base3])
                w1 = plsc.load_gather(bary_v, [base3 + 1])
                w2 = plsc.load_gather(bary_v, [base3 + 2])
                k16 = k * 16
                for p in range(16):
                    i = k16 + p
                    psplat = jnp.full((16,), p, jnp.int32)
                    s0 = _lane_bcast(w0, psplat)
                    s1 = _lane_bcast(w1, psplat)
                    s2 = _lane_bcast(w2, psplat)
                    acc = s0 * e0_v[i] + s1 * e1_v[i] + s2 * e2_v[i]
                    out_v[pl.ds(i * OUT_D, 16)] = acc
                base19 = rows * OUT_D
                for d in range(3):
                    xc = plsc.load_gather(x_v, [base3 + d])
                    plsc.store_scatter(out_v, [base19 + EMB + d], xc)
                return carry2

            lax.fori_loop(0, NBLK, blk, 0)

            pltpu.sync_copy(out_v, out_hbm.at[pl.ds(o * OUT_D, CH * OUT_D)])

        return carry

    lax.fori_loop(0, MAX_CH_W, chunk_body, 0)


def kernel(x, face_idxs, barycentrics, embeddings, faces):
    out_flat = _encode(x.reshape(-1), face_idxs, barycentrics.reshape(-1),
                       embeddings, faces.reshape(-1))
    return out_flat.reshape(N_POINTS, OUT_D)


# speedup vs baseline: 6.8045x; 6.8045x over previous
import functools

import jax
import jax.numpy as jnp
from jax import lax
from jax.experimental import pallas as pl
from jax.experimental.pallas import tpu as pltpu
from jax.experimental.pallas import tpu_sc as plsc

N_POINTS = 1_000_000
EMB = 16
OUT_D = EMB + 3
CH = 800
NCHUNK = N_POINTS // CH
NW = 32
MAX_CH_W = -(-NCHUNK // NW)
NBLK = CH // 16

_mesh = plsc.VectorSubcoreMesh(core_axis_name="c", subcore_axis_name="s")

_BCAST_DNUMS = lax.GatherDimensionNumbers(
    offset_dims=(), collapsed_slice_dims=(0,), start_index_map=(0,))


def _lane_bcast(vec, idx_splat):
    return lax.gather(vec, idx_splat[:, None], _BCAST_DNUMS, (1,),
                      mode=lax.GatherScatterMode.PROMISE_IN_BOUNDS)


@functools.partial(
    pl.kernel,
    mesh=_mesh,
    compiler_params=pltpu.CompilerParams(
        needs_layout_passes=False, use_tc_tiling_on_sc=False),
    out_type=jax.ShapeDtypeStruct((N_POINTS * OUT_D,), jnp.float32),
    scratch_types=[
        pltpu.VMEM((CH,), jnp.int32),
        pltpu.VMEM((CH,), jnp.int32),
        pltpu.VMEM((CH,), jnp.int32),
        pltpu.VMEM((CH,), jnp.int32),
        pltpu.VMEM((CH, EMB), jnp.float32),
        pltpu.VMEM((CH, EMB), jnp.float32),
        pltpu.VMEM((CH, EMB), jnp.float32),
        pltpu.VMEM((CH,), jnp.float32),
        pltpu.VMEM((CH,), jnp.float32),
        pltpu.VMEM((CH,), jnp.float32),
        pltpu.VMEM((CH,), jnp.float32),
        pltpu.VMEM((CH,), jnp.float32),
        pltpu.VMEM((CH,), jnp.float32),
        pltpu.VMEM((CH * OUT_D,), jnp.float32),
        pltpu.SemaphoreType.DMA,
    ],
)
def _encode(fidx_hbm, f0_hbm, f1_hbm, f2_hbm, b0_hbm, b1_hbm, b2_hbm,
            x0_hbm, x1_hbm, x2_hbm, emb_hbm, out_hbm,
            fidx_v, v0_v, v1_v, v2_v, e0_v, e1_v, e2_v,
            b0_v, b1_v, b2_v, x0_v, x1_v, x2_v, out_v, sem):
    wid = lax.axis_index("s") * 2 + lax.axis_index("c")

    def chunk_body(j, carry):
        c = j * NW + wid

        @pl.when(c < NCHUNK)
        def _():
            o = c * CH
            sl = pl.ds(o, CH)
            cps = [
                pltpu.async_copy(fidx_hbm.at[sl], fidx_v, sem),
                pltpu.async_copy(b0_hbm.at[sl], b0_v, sem),
                pltpu.async_copy(b1_hbm.at[sl], b1_v, sem),
                pltpu.async_copy(b2_hbm.at[sl], b2_v, sem),
                pltpu.async_copy(x0_hbm.at[sl], x0_v, sem),
                pltpu.async_copy(x1_hbm.at[sl], x1_v, sem),
                pltpu.async_copy(x2_hbm.at[sl], x2_v, sem),
            ]
            for cp in cps:
                cp.wait()

            cpa = pltpu.async_copy(f0_hbm.at[fidx_v], v0_v, sem)
            cpb = pltpu.async_copy(f1_hbm.at[fidx_v], v1_v, sem)
            cpc = pltpu.async_copy(f2_hbm.at[fidx_v], v2_v, sem)
            cpa.wait()
            cpb.wait()
            cpc.wait()

            cp0 = pltpu.async_copy(emb_hbm.at[v0_v], e0_v, sem)
            cp1 = pltpu.async_copy(emb_hbm.at[v1_v], e1_v, sem)
            cp2 = pltpu.async_copy(emb_hbm.at[v2_v], e2_v, sem)
            cp0.wait()
            cp1.wait()
            cp2.wait()

            def blk(k, carry2):
                ksl = pl.ds(k * 16, 16)
                w0 = b0_v[ksl]
                w1 = b1_v[ksl]
                w2 = b2_v[ksl]
                k16 = k * 16
                for p in range(16):
                    i = k16 + p
                    psplat = jnp.full((16,), p, jnp.int32)
                    s0 = _lane_bcast(w0, psplat)
                    s1 = _lane_bcast(w1, psplat)
                    s2 = _lane_bcast(w2, psplat)
                    acc = s0 * e0_v[i] + s1 * e1_v[i] + s2 * e2_v[i]
                    out_v[pl.ds(i * OUT_D, EMB)] = acc
                base19 = (lax.iota(jnp.int32, 16) + k16) * OUT_D
                plsc.store_scatter(out_v, [base19 + EMB], x0_v[ksl])
                plsc.store_scatter(out_v, [base19 + EMB + 1], x1_v[ksl])
                plsc.store_scatter(out_v, [base19 + EMB + 2], x2_v[ksl])
                return carry2

            lax.fori_loop(0, NBLK, blk, 0)

            pltpu.sync_copy(out_v, out_hbm.at[pl.ds(o * OUT_D, CH * OUT_D)])

        return carry

    lax.fori_loop(0, MAX_CH_W, chunk_body, 0)


def kernel(x, face_idxs, barycentrics, embeddings, faces):
    out_flat = _encode(
        face_idxs,
        faces[:, 0], faces[:, 1], faces[:, 2],
        barycentrics[:, 0], barycentrics[:, 1], barycentrics[:, 2],
        x[:, 0], x[:, 1], x[:, 2],
        embeddings)
    return out_flat.reshape(N_POINTS, OUT_D)


# speedup vs baseline: 6.9999x; 1.0287x over previous
import functools

import jax
import jax.numpy as jnp
from jax import lax
from jax.experimental import pallas as pl
from jax.experimental.pallas import tpu as pltpu
from jax.experimental.pallas import tpu_sc as plsc

N_POINTS = 1_000_000
EMB = 16
OUT_D = EMB + 3
CH = 800
NCHUNK = N_POINTS // CH
NW = 32
MAX_CH_W = -(-NCHUNK // NW)
NBLK = CH // 16

_mesh = plsc.VectorSubcoreMesh(core_axis_name="c", subcore_axis_name="s")

_BCAST_DNUMS = lax.GatherDimensionNumbers(
    offset_dims=(), collapsed_slice_dims=(0,), start_index_map=(0,))


def _lane_bcast(vec, idx_splat):
    return lax.gather(vec, idx_splat[:, None], _BCAST_DNUMS, (1,),
                      mode=lax.GatherScatterMode.PROMISE_IN_BOUNDS)


def _in_set():
    return [pltpu.VMEM((CH,), jnp.int32)] + [
        pltpu.VMEM((CH,), jnp.float32) for _ in range(6)]


@functools.partial(
    pl.kernel,
    mesh=_mesh,
    compiler_params=pltpu.CompilerParams(
        needs_layout_passes=False, use_tc_tiling_on_sc=False),
    out_type=jax.ShapeDtypeStruct((N_POINTS * OUT_D,), jnp.float32),
    scratch_types=(
        _in_set() + _in_set()
        + [pltpu.VMEM((CH,), jnp.int32)] * 3
        + [pltpu.VMEM((CH, EMB), jnp.float32)] * 6
        + [pltpu.VMEM((CH * OUT_D,), jnp.float32)] * 2
        + [pltpu.SemaphoreType.DMA] * 4
    ),
)
def _encode(fidx_hbm, f0_hbm, f1_hbm, f2_hbm, b0_hbm, b1_hbm, b2_hbm,
            x0_hbm, x1_hbm, x2_hbm, emb_hbm, out_hbm,
            fi0, b00, b10, b20, x00, x10, x20,
            fi1, b01, b11, b21, x01, x11, x21,
            v0_v, v1_v, v2_v,
            e00, e10, e20, e01, e11, e21,
            ov0, ov1, sem_a, sem_b, sem_e, sem_o):
    ins = ((fi0, b00, b10, b20, x00, x10, x20),
           (fi1, b01, b11, b21, x01, x11, x21))
    es = ((e00, e10, e20), (e01, e11, e21))
    ovs = (ov0, ov1)
    vs = (v0_v, v1_v, v2_v)
    fhbms = (f0_hbm, f1_hbm, f2_hbm)
    ihbms = (fidx_hbm, b0_hbm, b1_hbm, b2_hbm, x0_hbm, x1_hbm, x2_hbm)

    wid = lax.axis_index("s") * 2 + lax.axis_index("c")
    nj = jnp.where(wid < NCHUNK - (MAX_CH_W - 1) * NW, MAX_CH_W, MAX_CH_W - 1)

    def issue_a(j, p):
        sl = pl.ds((j * NW + wid) * CH, CH)
        return [pltpu.async_copy(h.at[sl], d, sem_a)
                for h, d in zip(ihbms, ins[p])]

    def issue_b(p):
        return [pltpu.async_copy(h.at[ins[p][0]], d, sem_b)
                for h, d in zip(fhbms, vs)]

    def issue_c(p):
        return [pltpu.async_copy(emb_hbm.at[v], e, sem_e)
                for v, e in zip(vs, es[p])]

    def drain_c(p):
        for e in es[p]:
            pltpu.make_async_copy(emb_hbm.at[pl.ds(0, CH)], e, sem_e).wait()

    def drain_o(p):
        pltpu.make_async_copy(
            ovs[p], out_hbm.at[pl.ds(0, CH * OUT_D)], sem_o).wait()

    def compute(j, p):
        e0_v, e1_v, e2_v = es[p]
        _, b0_v, b1_v, b2_v, x0_v, x1_v, x2_v = ins[p]
        out_v = ovs[p]

        def blk(k, carry2):
            ksl = pl.ds(k * 16, 16)
            w0 = b0_v[ksl]
            w1 = b1_v[ksl]
            w2 = b2_v[ksl]
            k16 = k * 16
            for pt in range(16):
                i = k16 + pt
                psplat = jnp.full((16,), pt, jnp.int32)
                s0 = _lane_bcast(w0, psplat)
                s1 = _lane_bcast(w1, psplat)
                s2 = _lane_bcast(w2, psplat)
                acc = s0 * e0_v[i] + s1 * e1_v[i] + s2 * e2_v[i]
                out_v[pl.ds(i * OUT_D, EMB)] = acc
            base19 = (lax.iota(jnp.int32, 16) + k16) * OUT_D
            plsc.store_scatter(out_v, [base19 + EMB], x0_v[ksl])
            plsc.store_scatter(out_v, [base19 + EMB + 1], x1_v[ksl])
            plsc.store_scatter(out_v, [base19 + EMB + 2], x2_v[ksl])
            return carry2

        lax.fori_loop(0, NBLK, blk, 0)
        pltpu.async_copy(
            out_v, out_hbm.at[pl.ds((j * NW + wid) * CH * OUT_D, CH * OUT_D)],
            sem_o)

    for cp in issue_a(0, 0):
        cp.wait()
    for cp in issue_b(0):
        cp.wait()
    issue_c(0)

    def body(i, carry):
        for q in (0, 1):
            j = i * 2 + q
            p = q
            np_ = 1 - q

            @pl.when(j < nj)
            def _():
                drain_c(p)

                @pl.when(j > 0)
                def _():
                    drain_o(np_)

                nxt = j + 1 < nj

                @pl.when(nxt)
                def _():
                    issue_a(j + 1, np_)

                compute(j, p)

                @pl.when(nxt)
                def _():
                    for h, d in zip(ihbms, ins[np_]):
                        pltpu.make_async_copy(
                            h.at[pl.ds(0, CH)], d, sem_a).wait()
                    for cp in issue_b(np_):
                        cp.wait()
                    issue_c(np_)

        return carry

    lax.fori_loop(0, MAX_CH_W // 2, body, 0)

    @pl.when((nj % 2) == 1)
    def _():
        drain_o(0)

    @pl.when((nj % 2) == 0)
    def _():
        drain_o(1)


def kernel(x, face_idxs, barycentrics, embeddings, faces):
    out_flat = _encode(
        face_idxs,
        faces[:, 0], faces[:, 1], faces[:, 2],
        barycentrics[:, 0], barycentrics[:, 1], barycentrics[:, 2],
        x[:, 0], x[:, 1], x[:, 2],
        embeddings)
    return out_flat.reshape(N_POINTS, OUT_D)
